# Initial kernel scaffold; baseline (speedup 1.0000x reference)
import functools

import jax
import jax.numpy as jnp
from jax import lax
from jax.experimental import pallas as pl
from jax.experimental.pallas import tpu as pltpu
from jax.experimental.pallas import tpu_sc as plsc

N_NODES = 100000
B = 320000
D_HALF = 128
NUM_BINS = 300

NC = 2
NS = 16
LANES = 16
NW = NC * NS

CHUNK = 80
B_PER_W = B // NW
N_CHUNKS = B_PER_W // CHUNK
BOUNDS_PAD = 320


def _body(mem_hbm, feat_hbm, emb_hbm, bounds_hbm, src_hbm, ivl_hbm, out_hbm,
          bounds_v, idx_v, glo_v, ghi_v, bix_v, ivl_v, lo_buf, hi_buf, sem):
    wid = lax.axis_index("s") * NC + lax.axis_index("c")
    base = wid * B_PER_W

    pltpu.sync_copy(bounds_hbm, bounds_v)

    def chunk_step(c, carry):
        cbase = pl.multiple_of(base + c * CHUNK, 8)

        pltpu.sync_copy(src_hbm.at[pl.ds(cbase, CHUNK)], idx_v)
        pltpu.sync_copy(ivl_hbm.at[pl.ds(cbase, CHUNK)], ivl_v)

        for j in range(CHUNK // LANES):
            sl = pl.ds(j * LANES, LANES)
            sid = idx_v[sl]
            glo_v[sl] = sid * 2
            ghi_v[sl] = sid * 2 + 1

            x = ivl_v[sl]
            cnt = jnp.zeros((LANES,), jnp.int32)
            for bit in (256, 128, 64, 32, 16, 8, 4, 2, 1):
                probe = cnt + (bit - 1)
                probe_c = jnp.minimum(probe, NUM_BINS)
                bv = plsc.load_gather(bounds_v, [probe_c])
                take = jnp.logical_and(bv < x, probe <= NUM_BINS)
                cnt = jnp.where(take, cnt + bit, cnt)
            bix_v[sl] = jnp.clip(cnt - 1, 0, NUM_BINS - 1)

        a1 = pltpu.async_copy(feat_hbm.at[idx_v], lo_buf, sem)
        a2 = pltpu.async_copy(emb_hbm.at[bix_v], hi_buf, sem)
        a1.wait()
        a2.wait()
        a3 = pltpu.async_copy(mem_hbm.at[glo_v], lo_buf, sem, add=True)
        a4 = pltpu.async_copy(mem_hbm.at[ghi_v], hi_buf, sem, add=True)
        a3.wait()
        a4.wait()

        pltpu.sync_copy(lo_buf, out_hbm.at[pl.ds(cbase, CHUNK), 0])
        pltpu.sync_copy(hi_buf, out_hbm.at[pl.ds(cbase, CHUNK), 1])
        return carry

    lax.fori_loop(0, N_CHUNKS, chunk_step, 0)


@jax.jit
def _run(mem_flat, node_features, emb_table, bounds_pad, src, intervals):
    fn = pl.kernel(
        _body,
        out_type=jax.ShapeDtypeStruct((B, 2, D_HALF), jnp.float32),
        mesh=plsc.VectorSubcoreMesh(
            core_axis_name="c", subcore_axis_name="s",
            num_cores=NC, num_subcores=NS),
        scratch_types=[
            pltpu.VMEM((BOUNDS_PAD,), jnp.float32),
            pltpu.VMEM((CHUNK,), jnp.int32),
            pltpu.VMEM((CHUNK,), jnp.int32),
            pltpu.VMEM((CHUNK,), jnp.int32),
            pltpu.VMEM((CHUNK,), jnp.int32),
            pltpu.VMEM((CHUNK,), jnp.float32),
            pltpu.VMEM((CHUNK, D_HALF), jnp.float32),
            pltpu.VMEM((CHUNK, D_HALF), jnp.float32),
            pltpu.SemaphoreType.DMA,
        ],
    )
    return fn(mem_flat, node_features, emb_table, bounds_pad, src, intervals)


def kernel(memory, node_features, emb_table, bin_boundaries, time_w, time_b,
           source_nodes, timestamps, intervals, route_len, n_layers):
    mem_flat = memory.reshape(2 * N_NODES, D_HALF)
    bounds_pad = jnp.concatenate(
        [bin_boundaries.astype(jnp.float32),
         jnp.full((BOUNDS_PAD - NUM_BINS - 1,), jnp.inf, jnp.float32)])
    src = source_nodes.astype(jnp.int32)
    out = _run(mem_flat, node_features, emb_table, bounds_pad, src,
               intervals.astype(jnp.float32))
    return out.reshape(B, 2 * D_HALF)


# speedup vs baseline: 16.7607x; 16.7607x over previous
import functools

import jax
import jax.numpy as jnp
from jax import lax
from jax.experimental import pallas as pl
from jax.experimental.pallas import tpu as pltpu
from jax.experimental.pallas import tpu_sc as plsc

N_NODES = 100000
B = 320000
D_HALF = 128
NUM_BINS = 300

NC = 2
NS = 16
LANES = 16
NW = NC * NS

CHUNK = 80
B_PER_W = B // NW
N_CHUNKS = B_PER_W // CHUNK
BOUNDS_PAD = 320


def _body(mem_hbm, feat_hbm, emb_hbm, bounds_hbm, src_hbm, ivl_hbm, out_hbm,
          bounds_v, idx_v, glo_v, ghi_v, bix_v, ivl_v, lo_buf, hi_buf, sem):
    wid = lax.axis_index("s") * NC + lax.axis_index("c")
    base = wid * B_PER_W

    pltpu.sync_copy(bounds_hbm, bounds_v)

    def chunk_step(c, carry):
        cbase = pl.multiple_of(base + c * CHUNK, 8)

        pltpu.sync_copy(src_hbm.at[pl.ds(cbase, CHUNK)], idx_v)
        pltpu.sync_copy(ivl_hbm.at[pl.ds(cbase, CHUNK)], ivl_v)

        for j in range(CHUNK // LANES):
            sl = pl.ds(j * LANES, LANES)
            sid = idx_v[sl]
            glo_v[sl] = sid * 2
            ghi_v[sl] = sid * 2 + 1

            x = ivl_v[sl]
            cnt = jnp.zeros((LANES,), jnp.int32)
            for bit in (256, 128, 64, 32, 16, 8, 4, 2, 1):
                probe = cnt + (bit - 1)
                probe_c = jnp.minimum(probe, NUM_BINS)
                bv = plsc.load_gather(bounds_v, [probe_c])
                take = jnp.logical_and(bv < x, probe <= NUM_BINS)
                cnt = jnp.where(take, cnt + bit, cnt)
            bix_v[sl] = jnp.clip(cnt - 1, 0, NUM_BINS - 1)

        a1 = pltpu.async_copy(feat_hbm.at[idx_v], lo_buf, sem)
        a2 = pltpu.async_copy(emb_hbm.at[bix_v], hi_buf, sem)
        a1.wait()
        a2.wait()
        a3 = pltpu.async_copy(mem_hbm.at[glo_v], lo_buf, sem, add=True)
        a4 = pltpu.async_copy(mem_hbm.at[ghi_v], hi_buf, sem, add=True)
        a3.wait()
        a4.wait()

        pltpu.sync_copy(lo_buf, out_hbm.at[pl.ds(cbase, CHUNK), 0])
        pltpu.sync_copy(hi_buf, out_hbm.at[pl.ds(cbase, CHUNK), 1])
        return carry

    lax.fori_loop(0, N_CHUNKS, chunk_step, 0)


@jax.jit
def _run(mem_flat, node_features, emb_table, bounds_pad, src, intervals):
    fn = pl.kernel(
        _body,
        out_type=jax.ShapeDtypeStruct((B, 2, D_HALF), jnp.float32),
        mesh=plsc.VectorSubcoreMesh(
            core_axis_name="c", subcore_axis_name="s",
            num_cores=NC, num_subcores=NS),
        scratch_types=[
            pltpu.VMEM((BOUNDS_PAD,), jnp.float32),
            pltpu.VMEM((CHUNK,), jnp.int32),
            pltpu.VMEM((CHUNK,), jnp.int32),
            pltpu.VMEM((CHUNK,), jnp.int32),
            pltpu.VMEM((CHUNK,), jnp.int32),
            pltpu.VMEM((CHUNK,), jnp.float32),
            pltpu.VMEM((CHUNK, D_HALF), jnp.float32),
            pltpu.VMEM((CHUNK, D_HALF), jnp.float32),
            pltpu.SemaphoreType.DMA,
        ],
        compiler_params=pltpu.CompilerParams(needs_layout_passes=False),
    )
    return fn(mem_flat, node_features, emb_table, bounds_pad, src, intervals)


def kernel(memory, node_features, emb_table, bin_boundaries, time_w, time_b,
           source_nodes, timestamps, intervals, route_len, n_layers):
    mem_flat = memory.reshape(2 * N_NODES, D_HALF)
    bounds_pad = jnp.concatenate(
        [bin_boundaries.astype(jnp.float32),
         jnp.full((BOUNDS_PAD - NUM_BINS - 1,), jnp.inf, jnp.float32)])
    src = source_nodes.astype(jnp.int32)
    out = _run(mem_flat, node_features, emb_table, bounds_pad, src,
               intervals.astype(jnp.float32))
    return out.reshape(B, 2 * D_HALF)


# speedup vs baseline: 20.9854x; 1.2521x over previous
import jax
import jax.numpy as jnp
from jax import lax
from jax.experimental import pallas as pl
from jax.experimental.pallas import tpu as pltpu
from jax.experimental.pallas import tpu_sc as plsc

N_NODES = 100000
B = 320000
D_HALF = 128
NUM_BINS = 300

NC = 2
NS = 16
LANES = 16
NW = NC * NS

CHUNK = 80
B_PER_W = B // NW
N_CHUNKS = B_PER_W // CHUNK
N_PAIRS = N_CHUNKS // 2
BOUNDS_PAD = 320

_SEARCH_BITS = (256, 128, 64, 32, 16, 8, 4, 2, 1)


def _body(mem_hbm, feat_hbm, emb_hbm, bounds_hbm, src_hbm, ivl_hbm, out_hbm,
          bounds_v, ids_v, ivl_v, glo_v, ghi_v, bix_v,
          lo_a, hi_a, lo_b, hi_b, sem_g, sem_a, sem_o):
    wid = lax.axis_index("s") * NC + lax.axis_index("c")
    base = wid * B_PER_W

    pltpu.sync_copy(bounds_hbm, bounds_v)
    pltpu.sync_copy(src_hbm.at[pl.ds(base, B_PER_W)], ids_v)
    pltpu.sync_copy(ivl_hbm.at[pl.ds(base, B_PER_W)], ivl_v)

    def prep(j, carry):
        sl = pl.ds(pl.multiple_of(j * LANES, LANES), LANES)
        sid = ids_v[sl]
        glo_v[sl] = sid * 2
        ghi_v[sl] = sid * 2 + 1
        x = ivl_v[sl]
        cnt = jnp.zeros((LANES,), jnp.int32)
        for bit in _SEARCH_BITS:
            probe = cnt + (bit - 1)
            probe_c = jnp.minimum(probe, NUM_BINS)
            bv = plsc.load_gather(bounds_v, [probe_c])
            take = jnp.logical_and(bv < x, probe <= NUM_BINS)
            cnt = jnp.where(take, cnt + bit, cnt)
        bix_v[sl] = jnp.clip(cnt - 1, 0, NUM_BINS - 1)
        return carry

    lax.fori_loop(0, B_PER_W // LANES, prep, 0)

    def fire_gathers(s, lo, hi):
        g1 = pltpu.async_copy(feat_hbm.at[ids_v.at[s]], lo, sem_g)
        g2 = pltpu.async_copy(emb_hbm.at[bix_v.at[s]], hi, sem_g)
        return g1, g2

    def fire_adds(s, lo, hi):
        a1 = pltpu.async_copy(mem_hbm.at[glo_v.at[s]], lo, sem_a, add=True)
        a2 = pltpu.async_copy(mem_hbm.at[ghi_v.at[s]], hi, sem_a, add=True)
        return a1, a2

    def fire_out(cb, lo, hi):
        pltpu.async_copy(lo, out_hbm.at[pl.ds(cb, CHUNK), 0], sem_o)
        pltpu.async_copy(hi, out_hbm.at[pl.ds(cb, CHUNK), 1], sem_o)

    def drain_out(lo, hi):
        pltpu.make_async_copy(lo, out_hbm.at[pl.ds(0, CHUNK), 0], sem_o).wait()
        pltpu.make_async_copy(hi, out_hbm.at[pl.ds(0, CHUNK), 1], sem_o).wait()

    def pair(i, carry):
        c0 = i * 2
        off0 = pl.multiple_of(c0 * CHUNK, 16)
        off1 = pl.multiple_of(c0 * CHUNK + CHUNK, 16)
        s0 = pl.ds(off0, CHUNK)
        s1 = pl.ds(off1, CHUNK)

        @pl.when(i != 0)
        def _():
            drain_out(lo_a, hi_a)

        ga1, ga2 = fire_gathers(s0, lo_a, hi_a)

        @pl.when(i != 0)
        def _():
            drain_out(lo_b, hi_b)

        gb1, gb2 = fire_gathers(s1, lo_b, hi_b)
        ga1.wait()
        ga2.wait()
        aa1, aa2 = fire_adds(s0, lo_a, hi_a)
        gb1.wait()
        gb2.wait()
        ab1, ab2 = fire_adds(s1, lo_b, hi_b)
        aa1.wait()
        aa2.wait()
        fire_out(base + off0, lo_a, hi_a)
        ab1.wait()
        ab2.wait()
        fire_out(base + off1, lo_b, hi_b)
        return carry

    lax.fori_loop(0, N_PAIRS, pair, 0)
    drain_out(lo_a, hi_a)
    drain_out(lo_b, hi_b)

    toff = pl.multiple_of((N_CHUNKS - 1) * CHUNK, 16)
    st = pl.ds(toff, CHUNK)
    g1, g2 = fire_gathers(st, lo_a, hi_a)
    g1.wait()
    g2.wait()
    a1, a2 = fire_adds(st, lo_a, hi_a)
    a1.wait()
    a2.wait()
    pltpu.sync_copy(lo_a, out_hbm.at[pl.ds(base + toff, CHUNK), 0])
    pltpu.sync_copy(hi_a, out_hbm.at[pl.ds(base + toff, CHUNK), 1])


@jax.jit
def _run(mem_flat, node_features, emb_table, bounds_pad, src, intervals):
    fn = pl.kernel(
        _body,
        out_type=jax.ShapeDtypeStruct((B, 2, D_HALF), jnp.float32),
        mesh=plsc.VectorSubcoreMesh(
            core_axis_name="c", subcore_axis_name="s",
            num_cores=NC, num_subcores=NS),
        scratch_types=[
            pltpu.VMEM((BOUNDS_PAD,), jnp.float32),
            pltpu.VMEM((B_PER_W,), jnp.int32),
            pltpu.VMEM((B_PER_W,), jnp.float32),
            pltpu.VMEM((B_PER_W,), jnp.int32),
            pltpu.VMEM((B_PER_W,), jnp.int32),
            pltpu.VMEM((B_PER_W,), jnp.int32),
            pltpu.VMEM((CHUNK, D_HALF), jnp.float32),
            pltpu.VMEM((CHUNK, D_HALF), jnp.float32),
            pltpu.VMEM((CHUNK, D_HALF), jnp.float32),
            pltpu.VMEM((CHUNK, D_HALF), jnp.float32),
            pltpu.SemaphoreType.DMA,
            pltpu.SemaphoreType.DMA,
            pltpu.SemaphoreType.DMA,
        ],
        compiler_params=pltpu.CompilerParams(needs_layout_passes=False),
    )
    return fn(mem_flat, node_features, emb_table, bounds_pad, src, intervals)


def kernel(memory, node_features, emb_table, bin_boundaries, time_w, time_b,
           source_nodes, timestamps, intervals, route_len, n_layers):
    mem_flat = memory.reshape(2 * N_NODES, D_HALF)
    bounds_pad = jnp.concatenate(
        [bin_boundaries.astype(jnp.float32),
         jnp.full((BOUNDS_PAD - NUM_BINS - 1,), jnp.inf, jnp.float32)])
    src = source_nodes.astype(jnp.int32)
    out = _run(mem_flat, node_features, emb_table, bounds_pad, src,
               intervals.astype(jnp.float32))
    return out.reshape(B, 2 * D_HALF)


# speedup vs baseline: 30.6728x; 1.4616x over previous
import jax
import jax.numpy as jnp
from jax import lax
from jax.experimental import pallas as pl
from jax.experimental.pallas import tpu as pltpu
from jax.experimental.pallas import tpu_sc as plsc

N_NODES = 100000
B = 320000
D_HALF = 128
NUM_BINS = 300

NC = 2
NS = 16
LANES = 16
NW = NC * NS

CHUNK = 80
B_PER_W = B // NW
N_CHUNKS = B_PER_W // CHUNK
N_PAIRS = N_CHUNKS // 2
BOUNDS_PAD = 320

_SEARCH_BITS = (256, 128, 64, 32, 16, 8, 4, 2, 1)


def _body(mem_hbm, feat_hbm, emb_hbm, bounds_hbm, src_hbm, ivl_hbm, out_hbm,
          bounds_v, ids_v, ivl_v, glo_v, ghi_v, bix_v,
          lo_a, hi_a, lo_b, hi_b, sem_g, sem_a, sem_o):
    wid = lax.axis_index("s") * NC + lax.axis_index("c")
    base = wid * B_PER_W

    pltpu.sync_copy(bounds_hbm, bounds_v)
    pltpu.sync_copy(src_hbm.at[pl.ds(base, B_PER_W)], ids_v)
    pltpu.sync_copy(ivl_hbm.at[pl.ds(base, B_PER_W)], ivl_v)

    def prep(j, carry):
        sl = pl.ds(pl.multiple_of(j * LANES, LANES), LANES)
        sid = ids_v[sl]
        glo_v[sl] = sid * 2
        ghi_v[sl] = sid * 2 + 1
        x = ivl_v[sl]
        cnt = jnp.zeros((LANES,), jnp.int32)
        for bit in _SEARCH_BITS:
            probe = cnt + (bit - 1)
            probe_c = jnp.minimum(probe, NUM_BINS)
            bv = plsc.load_gather(bounds_v, [probe_c])
            take = jnp.logical_and(bv < x, probe <= NUM_BINS)
            cnt = jnp.where(take, cnt + bit, cnt)
        bix_v[sl] = jnp.clip(cnt - 1, 0, NUM_BINS - 1)
        return carry

    lax.fori_loop(0, B_PER_W // LANES, prep, 0)

    def fire_gathers(s, lo, hi):
        g1 = pltpu.async_copy(feat_hbm.at[ids_v.at[s]], lo, sem_g)
        g2 = pltpu.async_copy(emb_hbm.at[bix_v.at[s]], hi, sem_g)
        return g1, g2

    def fire_adds(s, lo, hi):
        a1 = pltpu.async_copy(mem_hbm.at[glo_v.at[s]], lo, sem_a, add=True)
        a2 = pltpu.async_copy(mem_hbm.at[ghi_v.at[s]], hi, sem_a, add=True)
        return a1, a2

    def fire_out(cb, lo, hi):
        pltpu.async_copy(lo, out_hbm.at[pl.ds(cb, CHUNK), pl.ds(0, D_HALF)], sem_o)
        pltpu.async_copy(hi, out_hbm.at[pl.ds(cb, CHUNK), pl.ds(D_HALF, D_HALF)], sem_o)

    def drain_out(lo, hi):
        pltpu.make_async_copy(lo, out_hbm.at[pl.ds(0, CHUNK), pl.ds(0, D_HALF)], sem_o).wait()
        pltpu.make_async_copy(hi, out_hbm.at[pl.ds(0, CHUNK), pl.ds(D_HALF, D_HALF)], sem_o).wait()

    def pair(i, carry):
        c0 = i * 2
        off0 = pl.multiple_of(c0 * CHUNK, 16)
        off1 = pl.multiple_of(c0 * CHUNK + CHUNK, 16)
        s0 = pl.ds(off0, CHUNK)
        s1 = pl.ds(off1, CHUNK)

        @pl.when(i != 0)
        def _():
            drain_out(lo_a, hi_a)

        ga1, ga2 = fire_gathers(s0, lo_a, hi_a)

        @pl.when(i != 0)
        def _():
            drain_out(lo_b, hi_b)

        gb1, gb2 = fire_gathers(s1, lo_b, hi_b)
        ga1.wait()
        ga2.wait()
        aa1, aa2 = fire_adds(s0, lo_a, hi_a)
        gb1.wait()
        gb2.wait()
        ab1, ab2 = fire_adds(s1, lo_b, hi_b)
        aa1.wait()
        aa2.wait()
        fire_out(base + off0, lo_a, hi_a)
        ab1.wait()
        ab2.wait()
        fire_out(base + off1, lo_b, hi_b)
        return carry

    lax.fori_loop(0, N_PAIRS, pair, 0)
    drain_out(lo_a, hi_a)
    drain_out(lo_b, hi_b)

    toff = pl.multiple_of((N_CHUNKS - 1) * CHUNK, 16)
    st = pl.ds(toff, CHUNK)
    g1, g2 = fire_gathers(st, lo_a, hi_a)
    g1.wait()
    g2.wait()
    a1, a2 = fire_adds(st, lo_a, hi_a)
    a1.wait()
    a2.wait()
    pltpu.sync_copy(lo_a, out_hbm.at[pl.ds(base + toff, CHUNK), pl.ds(0, D_HALF)])
    pltpu.sync_copy(hi_a, out_hbm.at[pl.ds(base + toff, CHUNK), pl.ds(D_HALF, D_HALF)])


@jax.jit
def _run(mem_flat, node_features, emb_table, bounds_pad, src, intervals):
    fn = pl.kernel(
        _body,
        out_type=jax.ShapeDtypeStruct((B, 2 * D_HALF), jnp.float32),
        mesh=plsc.VectorSubcoreMesh(
            core_axis_name="c", subcore_axis_name="s",
            num_cores=NC, num_subcores=NS),
        scratch_types=[
            pltpu.VMEM((BOUNDS_PAD,), jnp.float32),
            pltpu.VMEM((B_PER_W,), jnp.int32),
            pltpu.VMEM((B_PER_W,), jnp.float32),
            pltpu.VMEM((B_PER_W,), jnp.int32),
            pltpu.VMEM((B_PER_W,), jnp.int32),
            pltpu.VMEM((B_PER_W,), jnp.int32),
            pltpu.VMEM((CHUNK, D_HALF), jnp.float32),
            pltpu.VMEM((CHUNK, D_HALF), jnp.float32),
            pltpu.VMEM((CHUNK, D_HALF), jnp.float32),
            pltpu.VMEM((CHUNK, D_HALF), jnp.float32),
            pltpu.SemaphoreType.DMA,
            pltpu.SemaphoreType.DMA,
            pltpu.SemaphoreType.DMA,
        ],
        compiler_params=pltpu.CompilerParams(needs_layout_passes=False),
    )
    return fn(mem_flat, node_features, emb_table, bounds_pad, src, intervals)


def kernel(memory, node_features, emb_table, bin_boundaries, time_w, time_b,
           source_nodes, timestamps, intervals, route_len, n_layers):
    mem_flat = memory.reshape(2 * N_NODES, D_HALF)
    bounds_pad = jnp.concatenate(
        [bin_boundaries.astype(jnp.float32),
         jnp.full((BOUNDS_PAD - NUM_BINS - 1,), jnp.inf, jnp.float32)])
    src = source_nodes.astype(jnp.int32)
    return _run(mem_flat, node_features, emb_table, bounds_pad, src,
                intervals.astype(jnp.float32))
